# Initial kernel scaffold; baseline (speedup 1.0000x reference)
import jax
import jax.numpy as jnp
from jax.experimental import pallas as pl


def kernel(x, emb_weight):
    raise NotImplementedError("write your pallas kernel here")


# speedup vs baseline: 2.3616x; 2.3616x over previous
import jax
import jax.numpy as jnp
from jax.experimental import pallas as pl
from jax.experimental.pallas import tpu as pltpu
from jax.experimental.pallas import tpu_sc as plsc

D_MODEL = 128
SCALE = float(D_MODEL) ** 0.5
GATHER_WINDOW = 128


def _scale_table(table):
    rows = table.shape[0]
    block_rows = 2000
    grid = rows // block_rows

    def body(t_ref, o_ref):
        o_ref[...] = t_ref[...] * SCALE

    return pl.pallas_call(
        body,
        grid=(grid,),
        in_specs=[pl.BlockSpec((block_rows, D_MODEL), lambda i: (i, 0))],
        out_specs=pl.BlockSpec((block_rows, D_MODEL), lambda i: (i, 0)),
        out_shape=jax.ShapeDtypeStruct(table.shape, table.dtype),
    )(table)


def _sc_gather(table, indices):
    num_indices = indices.shape[1]
    mesh = plsc.VectorSubcoreMesh(core_axis_name="c", subcore_axis_name="s")

    @pl.kernel(
        out_type=jax.ShapeDtypeStruct((num_indices, D_MODEL), table.dtype),
        mesh=mesh,
    )
    def k(table_hbm, idx_hbm, out_hbm):
        def body(idx_vmem, out_vmem):
            pltpu.sync_copy(table_hbm.at[idx_vmem.at[0]], out_vmem)

        pltpu.emit_pipeline(
            body,
            grid=(num_indices // GATHER_WINDOW,),
            in_specs=[
                pl.BlockSpec((1, GATHER_WINDOW), index_map=lambda i: (0, i))
            ],
            out_specs=[
                pl.BlockSpec((GATHER_WINDOW, D_MODEL), index_map=lambda i: (i, 0))
            ],
            core_axis_name=("c", "s"),
            dimension_semantics=(pltpu.PARALLEL,),
        )(idx_hbm, out_hbm)

    return k(table, indices)


def kernel(x, emb_weight):
    scaled = _scale_table(emb_weight)
    flat_idx = x.reshape(1, -1).astype(jnp.int32)
    out = _sc_gather(scaled, flat_idx)
    return out.reshape(x.shape + (D_MODEL,))


# speedup vs baseline: 3.0688x; 1.2995x over previous
import jax
import jax.numpy as jnp
from jax.experimental import pallas as pl
from jax.experimental.pallas import tpu as pltpu
from jax.experimental.pallas import tpu_sc as plsc

D_MODEL = 128
SCALE = float(D_MODEL) ** 0.5
GATHER_WINDOW = 128


def _scale_table(table):
    rows = table.shape[0]
    block_rows = 2000
    grid = rows // block_rows

    def body(t_ref, o_ref):
        o_ref[...] = t_ref[...] * SCALE

    return pl.pallas_call(
        body,
        grid=(grid,),
        in_specs=[pl.BlockSpec((block_rows, D_MODEL), lambda i: (i, 0))],
        out_specs=pl.BlockSpec((block_rows, D_MODEL), lambda i: (i, 0)),
        out_shape=jax.ShapeDtypeStruct(table.shape, table.dtype),
    )(table)


B_BLK = 8


def _sc_gather(table, indices):
    batch, seq = indices.shape
    mesh = plsc.VectorSubcoreMesh(core_axis_name="c", subcore_axis_name="s")

    @pl.kernel(
        out_type=jax.ShapeDtypeStruct((batch, seq, D_MODEL), table.dtype),
        mesh=mesh,
    )
    def k(table_hbm, idx_hbm, out_hbm):
        def body(idx_vmem, out_vmem):
            for b in range(B_BLK):
                pltpu.sync_copy(
                    table_hbm.at[idx_vmem.at[b]], out_vmem.at[b]
                )

        pltpu.emit_pipeline(
            body,
            grid=(batch // B_BLK,),
            in_specs=[
                pl.BlockSpec((B_BLK, seq), index_map=lambda i: (i, 0))
            ],
            out_specs=[
                pl.BlockSpec(
                    (B_BLK, seq, D_MODEL), index_map=lambda i: (i, 0, 0)
                )
            ],
            core_axis_name=("c", "s"),
            dimension_semantics=(pltpu.PARALLEL,),
        )(idx_hbm, out_hbm)

    return k(table, indices)


def kernel(x, emb_weight):
    scaled = _scale_table(emb_weight)
    out = _sc_gather(scaled, x.astype(jnp.int32))
    return out
